# baseline (speedup 1.0000x reference)
import functools

import jax
import jax.numpy as jnp
from jax import lax
from jax.experimental import pallas as pl
from jax.experimental.pallas import tpu as pltpu
from jax.experimental.pallas import tpu_sc as plsc

H = 128
_NC = 2
_NS = 16
_NW = _NC * _NS


def _dot(a, b):
    return jnp.dot(a, b, preferred_element_type=jnp.float32,
                   precision=lax.Precision.DEFAULT)


def _make_mlp_body(n, m, residual, proj, block, nsplit):
    rows = block // nsplit

    def body(*refs):
        xs = refs[:n]
        adds = refs[n:n + m]
        w1s = refs[n + m:2 * n + m]
        rest = refs[2 * n + m:]
        if proj:
            b1, g, beta, w2, b2, wp, out, out2 = rest
        else:
            b1, g, beta, w2, b2, out = rest
        for s in range(nsplit):
            sl = pl.ds(s * rows, rows)
            acc = _dot(xs[0][sl, :], w1s[0][...])
            for i in range(1, n):
                acc = acc + _dot(xs[i][sl, :], w1s[i][...])
            for a in adds:
                acc = acc + a[sl, :]
            h = acc + b1[...][None]
            h = h * jax.nn.sigmoid(h)
            s1 = jnp.sum(h, axis=-1, keepdims=True)
            s2 = jnp.sum(h * h, axis=-1, keepdims=True)
            mu = s1 * (1.0 / H)
            var = s2 * (1.0 / H) - mu * mu
            t = lax.rsqrt(var + 1e-5)
            hn = (h - mu) * t * g[...][None] + beta[...][None]
            o = _dot(hn, w2[...]) + b2[...][None]
            if residual:
                o = xs[0][sl, :] + o
            out[sl, :] = o
            if proj:
                out2[sl, :] = _dot(o, wp[...])
    return body


def _tc_mlp(xs, p, residual=False, block=2000, w1_offsets=None,
            adds=(), extra_proj=None):
    n = len(xs)
    m = len(adds)
    N = xs[0].shape[0]
    assert N % block == 0, (N, block)
    if w1_offsets is None:
        w1_offsets = []
        off = 0
        for x in xs:
            w1_offsets.append(off)
            off += x.shape[1]
        assert off == p["W1"].shape[0]
    w1s = []
    for x, off in zip(xs, w1_offsets):
        d = x.shape[1]
        w1s.append(lax.slice(p["W1"], (off, 0), (off + d, H)))

    in_specs = []
    for x in xs:
        d = x.shape[1]
        in_specs.append(pl.BlockSpec((block, d), lambda i: (i, 0)))
    for _ in adds:
        in_specs.append(pl.BlockSpec((block, H), lambda i: (i, 0)))
    for x in xs:
        d = x.shape[1]
        in_specs.append(pl.BlockSpec((d, H), lambda i: (0, 0)))
    in_specs += [
        pl.BlockSpec((H,), lambda i: (0,)),
        pl.BlockSpec((H,), lambda i: (0,)),
        pl.BlockSpec((H,), lambda i: (0,)),
        pl.BlockSpec((H, H), lambda i: (0, 0)),
        pl.BlockSpec((H,), lambda i: (0,)),
    ]
    operands = [*xs, *adds, *w1s, p["b1"], p["g"], p["beta"], p["W2"],
                p["b2"]]
    if extra_proj is not None:
        in_specs.append(pl.BlockSpec((H, H), lambda i: (0, 0)))
        operands.append(extra_proj)
        out_specs = (pl.BlockSpec((block, H), lambda i: (i, 0)),
                     pl.BlockSpec((block, H), lambda i: (i, 0)))
        out_shape = (jax.ShapeDtypeStruct((N, H), jnp.float32),
                     jax.ShapeDtypeStruct((N, H), jnp.float32))
    else:
        out_specs = pl.BlockSpec((block, H), lambda i: (i, 0))
        out_shape = jax.ShapeDtypeStruct((N, H), jnp.float32)
    nsplit = 2 if block % 16 == 0 else 1
    return pl.pallas_call(
        _make_mlp_body(n, m, residual, extra_proj is not None, block, nsplit),
        grid=(N // block,),
        in_specs=in_specs,
        out_specs=out_specs,
        out_shape=out_shape,
        compiler_params=pltpu.CompilerParams(
            dimension_semantics=("arbitrary",)),
    )(*operands)


def _sc_gather_sum(p_tab, q_tab, sidx3, ridx3, E, K, C):
    mesh = plsc.VectorSubcoreMesh(core_axis_name="c", subcore_axis_name="s")

    def body(p_hbm, q_hbm, sidx_hbm, ridx_hbm, out_hbm,
             sidx_v, ridx_v, buf0, buf1, semp0, semp1, semq0, semq1):
        cid = lax.axis_index("c")
        sid = lax.axis_index("s")
        wid = sid * _NC + cid
        base = wid * (K * C)
        bufs = (buf0, buf1)
        semps = (semp0, semp1)
        semqs = (semq0, semq1)

        pltpu.sync_copy(sidx_hbm.at[wid], sidx_v)
        pltpu.sync_copy(ridx_hbm.at[wid], ridx_v)

        def start_p(j, slot):
            pltpu.async_copy(p_hbm.at[sidx_v.at[j]], bufs[slot], semps[slot])

        def wait_p(slot):
            pltpu.make_async_copy(p_hbm.at[sidx_v.at[0]], bufs[slot],
                                  semps[slot]).wait()

        def start_q(j, slot):
            pltpu.async_copy(q_hbm.at[ridx_v.at[j]], bufs[slot], semqs[slot],
                             add=True)

        def wait_q(slot):
            pltpu.make_async_copy(q_hbm.at[ridx_v.at[0]], bufs[slot],
                                  semqs[slot]).wait()

        def drain(j, slot):
            pltpu.sync_copy(bufs[slot], out_hbm.at[pl.ds(base + j * C, C)])

        start_p(0, 0)
        wait_p(0)
        start_q(0, 0)

        def pair(p, carry):
            a = 2 * p
            b = a + 1
            start_p(b, 1)
            wait_q(0)
            drain(a, 0)
            wait_p(1)
            start_q(b, 1)

            @pl.when(b + 1 < K)
            def _():
                start_p(b + 1, 0)

            wait_q(1)
            drain(b, 1)

            @pl.when(b + 1 < K)
            def _():
                wait_p(0)
                start_q(b + 1, 0)
            return carry
        lax.fori_loop(0, K // 2, pair, 0)
        if K % 2:
            wait_q(0)
            drain(K - 1, 0)

    k = pl.kernel(
        body,
        out_type=jax.ShapeDtypeStruct((E, H), jnp.float32),
        mesh=mesh,
        scratch_types=[
            pltpu.VMEM((K, C), jnp.int32),
            pltpu.VMEM((K, C), jnp.int32),
            pltpu.VMEM((C, H), jnp.float32),
            pltpu.VMEM((C, H), jnp.float32),
            pltpu.SemaphoreType.DMA,
            pltpu.SemaphoreType.DMA,
            pltpu.SemaphoreType.DMA,
            pltpu.SemaphoreType.DMA,
        ],
    )
    return k(p_tab, q_tab, sidx3, ridx3)


def _sc_scatter_add(msg, ridx3, Nmp, K, C, seed=None):
    mesh = plsc.VectorSubcoreMesh(core_axis_name="c", subcore_axis_name="s")
    zrows = Nmp // _NS
    ZB = 80
    assert zrows % ZB == 0 and zrows % 8 == 0

    def body(*refs):
        if seed is None:
            msg_hbm, ridx_hbm, out_hbm, idx_v, buf0, buf1, zbuf, acc, \
                sem0, sem1 = refs
        else:
            msg_hbm, ridx_hbm, seed_hbm, out_hbm, idx_v, buf0, buf1, acc, \
                sem0, sem1 = refs
        cid = lax.axis_index("c")
        sid = lax.axis_index("s")
        wid = sid * _NC + cid
        base = wid * (K * C)
        bufs = (buf0, buf1)
        sems = (sem0, sem1)

        pltpu.sync_copy(ridx_hbm.at[wid], idx_v)

        if seed is None:
            def zstep(r, carry):
                for c16 in range(H // 16):
                    zbuf[r, pl.ds(c16 * 16, 16)] = jnp.zeros((16,),
                                                             jnp.float32)
                return carry
            lax.fori_loop(0, ZB, zstep, 0)
            for t in range(zrows // ZB):
                pltpu.sync_copy(zbuf, acc.at[pl.ds(sid * zrows + t * ZB, ZB)])
        else:
            pltpu.sync_copy(seed_hbm.at[cid, pl.ds(sid * zrows, zrows)],
                            acc.at[pl.ds(sid * zrows, zrows)])
        plsc.subcore_barrier()

        def start(j, slot):
            pltpu.async_copy(msg_hbm.at[pl.ds(base + j * C, C)], bufs[slot],
                             sems[slot])

        def wait(slot):
            pltpu.make_async_copy(msg_hbm.at[pl.ds(base, C)], bufs[slot],
                                  sems[slot]).wait()

        def drain(j, slot):
            pltpu.sync_copy(bufs[slot], acc.at[idx_v.at[j]], add=True)

        start(0, 0)

        def pair(p, carry):
            a = 2 * p
            b = a + 1
            wait(0)
            start(b, 1)
            drain(a, 0)
            wait(1)

            @pl.when(b + 1 < K)
            def _():
                start(b + 1, 0)
            drain(b, 1)
            return carry
        lax.fori_loop(0, K // 2, pair, 0)
        if K % 2:
            wait(0)
            drain(K - 1, 0)
        plsc.subcore_barrier()

        pltpu.sync_copy(acc.at[pl.ds(sid * zrows, zrows)],
                        out_hbm.at[cid, pl.ds(sid * zrows, zrows)])

    scratch = [
        pltpu.VMEM((K, C), jnp.int32),
        pltpu.VMEM((C, H), jnp.float32),
        pltpu.VMEM((C, H), jnp.float32),
    ]
    if seed is None:
        scratch.append(pltpu.VMEM((ZB, H), jnp.float32))
    scratch += [
        pltpu.VMEM_SHARED((Nmp, H), jnp.float32),
        pltpu.SemaphoreType.DMA,
        pltpu.SemaphoreType.DMA,
    ]
    k = pl.kernel(
        body,
        out_type=jax.ShapeDtypeStruct((_NC, Nmp, H), jnp.float32),
        mesh=mesh,
        scratch_types=scratch,
    )
    if seed is None:
        return k(msg, ridx3)
    return k(msg, ridx3, seed)


def kernel(inputs, grid_structural, mesh_structural, M2M_edge_structural,
           G2M_edge_structural, M2G_edge_structural, senders, receivers,
           params):
    B_, Ng, _ = inputs.shape
    Nm = mesh_structural.shape[0]
    E = senders.shape[0]
    assert B_ == 1

    C = 40
    E1 = (E * 3 // 5) // (_NW * C) * (_NW * C)
    E2 = E - E1
    assert E2 % (_NW * C) == 0, (E, E1, E2)
    K1 = E1 // (_NW * C)
    K2 = E2 // (_NW * C)

    x_grid = inputs.reshape(Ng, inputs.shape[2])
    s32 = senders.astype(jnp.int32)
    r32 = receivers.astype(jnp.int32)
    sidx1 = lax.slice(s32, (0,), (E1,)).reshape(_NW, K1, C)
    ridx1 = lax.slice(r32, (0,), (E1,)).reshape(_NW, K1, C)
    sidx2 = lax.slice(s32, (E1,), (E,)).reshape(_NW, K2, C)
    ridx2 = lax.slice(r32, (E1,), (E,)).reshape(_NW, K2, C)

    w1_msg = params["G2M_message"]["W1"]
    w1_gs = lax.slice(w1_msg, (H, 0), (2 * H, H))
    w1_ms = lax.slice(w1_msg, (2 * H, 0), (3 * H, H))
    vG, P = _tc_mlp([x_grid, grid_structural], params["grid_node_embed"],
                    extra_proj=w1_gs)
    vM, Q = _tc_mlp([mesh_structural], params["mesh_node_embed"],
                    extra_proj=w1_ms)

    g1 = _sc_gather_sum(P, Q, sidx1, ridx1, E1, K1, C)
    eG2M = _tc_mlp([G2M_edge_structural], params["G2M_edge_embed"],
                   block=4000)
    eM2M = _tc_mlp([M2M_edge_structural], params["M2M_edge_embed"],
                   block=4000)
    eM2G = _tc_mlp([M2G_edge_structural], params["M2G_edge_embed"],
                   block=4000)

    g2 = _sc_gather_sum(P, Q, sidx2, ridx2, E2, K2, C)
    eG2M_1 = lax.slice(eG2M, (0, 0), (E1, H))
    eG2M_2 = lax.slice(eG2M, (E1, 0), (E, H))

    Nmp = ((Nm + 1280 - 1) // 1280) * 1280
    msg1 = _tc_mlp([eG2M_1], params["G2M_message"], residual=True,
                   block=4000, w1_offsets=[0], adds=[g1])
    part1 = _sc_scatter_add(msg1, ridx1, Nmp, K1, C)
    msg2 = _tc_mlp([eG2M_2], params["G2M_message"], residual=True,
                   block=4000, w1_offsets=[0], adds=[g2])
    part2 = _sc_scatter_add(msg2, ridx2, Nmp, K2, C, seed=part1)
    vG_new = _tc_mlp([vG], params["G_update"], residual=True)
    vM_new = _tc_mlp([vM, part2[0, :Nm], part2[1, :Nm]],
                     params["G2M_node_update"],
                     residual=True, w1_offsets=[0, H, H])

    return (vM_new[None], vG_new[None], eM2M, eM2G)


# speedup vs baseline: 1.0523x; 1.0523x over previous
import functools

import jax
import jax.numpy as jnp
from jax import lax
from jax.experimental import pallas as pl
from jax.experimental.pallas import tpu as pltpu
from jax.experimental.pallas import tpu_sc as plsc

H = 128
_NC = 2
_NS = 16
_NW = _NC * _NS


def _dot(a, b):
    return jnp.dot(a, b, preferred_element_type=jnp.float32,
                   precision=lax.Precision.DEFAULT)


def _make_mlp_body(n, m, residual, proj, block, nsplit):
    rows = block // nsplit

    def body(*refs):
        xs = refs[:n]
        adds = refs[n:n + m]
        w1s = refs[n + m:2 * n + m]
        rest = refs[2 * n + m:]
        if proj:
            b1, g, beta, w2, b2, wp, out, out2 = rest
        else:
            b1, g, beta, w2, b2, out = rest
        for s in range(nsplit):
            sl = pl.ds(s * rows, rows)
            acc = _dot(xs[0][sl, :], w1s[0][...])
            for i in range(1, n):
                acc = acc + _dot(xs[i][sl, :], w1s[i][...])
            for a in adds:
                acc = acc + a[sl, :]
            h = acc + b1[...][None]
            h = h * jax.nn.sigmoid(h)
            s1 = jnp.sum(h, axis=-1, keepdims=True)
            s2 = jnp.sum(h * h, axis=-1, keepdims=True)
            mu = s1 * (1.0 / H)
            var = s2 * (1.0 / H) - mu * mu
            t = lax.rsqrt(var + 1e-5)
            hn = (h - mu) * t * g[...][None] + beta[...][None]
            o = _dot(hn, w2[...]) + b2[...][None]
            if residual:
                o = xs[0][sl, :] + o
            out[sl, :] = o
            if proj:
                out2[sl, :] = _dot(o, wp[...])
    return body


def _tc_mlp(xs, p, residual=False, block=5000, w1_offsets=None,
            adds=(), extra_proj=None):
    n = len(xs)
    m = len(adds)
    N = xs[0].shape[0]
    assert N % block == 0, (N, block)
    if w1_offsets is None:
        w1_offsets = []
        off = 0
        for x in xs:
            w1_offsets.append(off)
            off += x.shape[1]
        assert off == p["W1"].shape[0]
    w1s = []
    for x, off in zip(xs, w1_offsets):
        d = x.shape[1]
        w1s.append(lax.slice(p["W1"], (off, 0), (off + d, H)))

    in_specs = []
    for x in xs:
        d = x.shape[1]
        in_specs.append(pl.BlockSpec((block, d), lambda i: (i, 0)))
    for _ in adds:
        in_specs.append(pl.BlockSpec((block, H), lambda i: (i, 0)))
    for x in xs:
        d = x.shape[1]
        in_specs.append(pl.BlockSpec((d, H), lambda i: (0, 0)))
    in_specs += [
        pl.BlockSpec((H,), lambda i: (0,)),
        pl.BlockSpec((H,), lambda i: (0,)),
        pl.BlockSpec((H,), lambda i: (0,)),
        pl.BlockSpec((H, H), lambda i: (0, 0)),
        pl.BlockSpec((H,), lambda i: (0,)),
    ]
    operands = [*xs, *adds, *w1s, p["b1"], p["g"], p["beta"], p["W2"],
                p["b2"]]
    if extra_proj is not None:
        in_specs.append(pl.BlockSpec((H, H), lambda i: (0, 0)))
        operands.append(extra_proj)
        out_specs = (pl.BlockSpec((block, H), lambda i: (i, 0)),
                     pl.BlockSpec((block, H), lambda i: (i, 0)))
        out_shape = (jax.ShapeDtypeStruct((N, H), jnp.float32),
                     jax.ShapeDtypeStruct((N, H), jnp.float32))
    else:
        out_specs = pl.BlockSpec((block, H), lambda i: (i, 0))
        out_shape = jax.ShapeDtypeStruct((N, H), jnp.float32)
    nsplit = 1
    return pl.pallas_call(
        _make_mlp_body(n, m, residual, extra_proj is not None, block, nsplit),
        grid=(N // block,),
        in_specs=in_specs,
        out_specs=out_specs,
        out_shape=out_shape,
        compiler_params=pltpu.CompilerParams(
            dimension_semantics=("arbitrary",)),
    )(*operands)


def _sc_gather_sum(p_tab, q_tab, sidx3, ridx3, E, K, C):
    mesh = plsc.VectorSubcoreMesh(core_axis_name="c", subcore_axis_name="s")

    def body(p_hbm, q_hbm, sidx_hbm, ridx_hbm, out_hbm,
             sidx_v, ridx_v, buf0, buf1, semp0, semp1, semq0, semq1):
        cid = lax.axis_index("c")
        sid = lax.axis_index("s")
        wid = sid * _NC + cid
        base = wid * (K * C)
        bufs = (buf0, buf1)
        semps = (semp0, semp1)
        semqs = (semq0, semq1)

        pltpu.sync_copy(sidx_hbm.at[wid], sidx_v)
        pltpu.sync_copy(ridx_hbm.at[wid], ridx_v)

        def start_p(j, slot):
            pltpu.async_copy(p_hbm.at[sidx_v.at[j]], bufs[slot], semps[slot])

        def wait_p(slot):
            pltpu.make_async_copy(p_hbm.at[sidx_v.at[0]], bufs[slot],
                                  semps[slot]).wait()

        def start_q(j, slot):
            pltpu.async_copy(q_hbm.at[ridx_v.at[j]], bufs[slot], semqs[slot],
                             add=True)

        def wait_q(slot):
            pltpu.make_async_copy(q_hbm.at[ridx_v.at[0]], bufs[slot],
                                  semqs[slot]).wait()

        def drain(j, slot):
            pltpu.sync_copy(bufs[slot], out_hbm.at[pl.ds(base + j * C, C)])

        start_p(0, 0)
        wait_p(0)
        start_q(0, 0)

        def pair(p, carry):
            a = 2 * p
            b = a + 1
            start_p(b, 1)
            wait_q(0)
            drain(a, 0)
            wait_p(1)
            start_q(b, 1)

            @pl.when(b + 1 < K)
            def _():
                start_p(b + 1, 0)

            wait_q(1)
            drain(b, 1)

            @pl.when(b + 1 < K)
            def _():
                wait_p(0)
                start_q(b + 1, 0)
            return carry
        lax.fori_loop(0, K // 2, pair, 0)
        if K % 2:
            wait_q(0)
            drain(K - 1, 0)

    k = pl.kernel(
        body,
        out_type=jax.ShapeDtypeStruct((E, H), jnp.float32),
        mesh=mesh,
        scratch_types=[
            pltpu.VMEM((K, C), jnp.int32),
            pltpu.VMEM((K, C), jnp.int32),
            pltpu.VMEM((C, H), jnp.float32),
            pltpu.VMEM((C, H), jnp.float32),
            pltpu.SemaphoreType.DMA,
            pltpu.SemaphoreType.DMA,
            pltpu.SemaphoreType.DMA,
            pltpu.SemaphoreType.DMA,
        ],
    )
    return k(p_tab, q_tab, sidx3, ridx3)


def _sc_scatter_add(msg, ridx3, Nmp, K, C, seed=None):
    mesh = plsc.VectorSubcoreMesh(core_axis_name="c", subcore_axis_name="s")
    zrows = Nmp // _NS
    ZB = 80
    assert zrows % ZB == 0 and zrows % 8 == 0

    def body(*refs):
        if seed is None:
            msg_hbm, ridx_hbm, out_hbm, idx_v, buf0, buf1, zbuf, acc, \
                sem0, sem1 = refs
        else:
            msg_hbm, ridx_hbm, seed_hbm, out_hbm, idx_v, buf0, buf1, acc, \
                sem0, sem1 = refs
        cid = lax.axis_index("c")
        sid = lax.axis_index("s")
        wid = sid * _NC + cid
        base = wid * (K * C)
        bufs = (buf0, buf1)
        sems = (sem0, sem1)

        pltpu.sync_copy(ridx_hbm.at[wid], idx_v)

        if seed is None:
            def zstep(r, carry):
                for c16 in range(H // 16):
                    zbuf[r, pl.ds(c16 * 16, 16)] = jnp.zeros((16,),
                                                             jnp.float32)
                return carry
            lax.fori_loop(0, ZB, zstep, 0)
            for t in range(zrows // ZB):
                pltpu.sync_copy(zbuf, acc.at[pl.ds(sid * zrows + t * ZB, ZB)])
        else:
            pltpu.sync_copy(seed_hbm.at[cid, pl.ds(sid * zrows, zrows)],
                            acc.at[pl.ds(sid * zrows, zrows)])
        plsc.subcore_barrier()

        def start(j, slot):
            pltpu.async_copy(msg_hbm.at[pl.ds(base + j * C, C)], bufs[slot],
                             sems[slot])

        def wait(slot):
            pltpu.make_async_copy(msg_hbm.at[pl.ds(base, C)], bufs[slot],
                                  sems[slot]).wait()

        def drain(j, slot):
            pltpu.sync_copy(bufs[slot], acc.at[idx_v.at[j]], add=True)

        start(0, 0)

        def pair(p, carry):
            a = 2 * p
            b = a + 1
            wait(0)
            start(b, 1)
            drain(a, 0)
            wait(1)

            @pl.when(b + 1 < K)
            def _():
                start(b + 1, 0)
            drain(b, 1)
            return carry
        lax.fori_loop(0, K // 2, pair, 0)
        if K % 2:
            wait(0)
            drain(K - 1, 0)
        plsc.subcore_barrier()

        pltpu.sync_copy(acc.at[pl.ds(sid * zrows, zrows)],
                        out_hbm.at[cid, pl.ds(sid * zrows, zrows)])

    scratch = [
        pltpu.VMEM((K, C), jnp.int32),
        pltpu.VMEM((C, H), jnp.float32),
        pltpu.VMEM((C, H), jnp.float32),
    ]
    if seed is None:
        scratch.append(pltpu.VMEM((ZB, H), jnp.float32))
    scratch += [
        pltpu.VMEM_SHARED((Nmp, H), jnp.float32),
        pltpu.SemaphoreType.DMA,
        pltpu.SemaphoreType.DMA,
    ]
    k = pl.kernel(
        body,
        out_type=jax.ShapeDtypeStruct((_NC, Nmp, H), jnp.float32),
        mesh=mesh,
        scratch_types=scratch,
    )
    if seed is None:
        return k(msg, ridx3)
    return k(msg, ridx3, seed)


def kernel(inputs, grid_structural, mesh_structural, M2M_edge_structural,
           G2M_edge_structural, M2G_edge_structural, senders, receivers,
           params):
    B_, Ng, _ = inputs.shape
    Nm = mesh_structural.shape[0]
    E = senders.shape[0]
    assert B_ == 1

    C = 40
    E1 = (E * 3 // 5) // (_NW * C) * (_NW * C)
    E2 = E - E1
    assert E2 % (_NW * C) == 0, (E, E1, E2)
    K1 = E1 // (_NW * C)
    K2 = E2 // (_NW * C)

    x_grid = inputs.reshape(Ng, inputs.shape[2])
    s32 = senders.astype(jnp.int32)
    r32 = receivers.astype(jnp.int32)
    sidx1 = lax.slice(s32, (0,), (E1,)).reshape(_NW, K1, C)
    ridx1 = lax.slice(r32, (0,), (E1,)).reshape(_NW, K1, C)
    sidx2 = lax.slice(s32, (E1,), (E,)).reshape(_NW, K2, C)
    ridx2 = lax.slice(r32, (E1,), (E,)).reshape(_NW, K2, C)

    w1_msg = params["G2M_message"]["W1"]
    w1_gs = lax.slice(w1_msg, (H, 0), (2 * H, H))
    w1_ms = lax.slice(w1_msg, (2 * H, 0), (3 * H, H))
    vG, P = _tc_mlp([x_grid, grid_structural], params["grid_node_embed"],
                    extra_proj=w1_gs)
    vM, Q = _tc_mlp([mesh_structural], params["mesh_node_embed"],
                    extra_proj=w1_ms)

    g1 = _sc_gather_sum(P, Q, sidx1, ridx1, E1, K1, C)
    eG2M = _tc_mlp([G2M_edge_structural], params["G2M_edge_embed"])
    eM2M = _tc_mlp([M2M_edge_structural], params["M2M_edge_embed"])
    eM2G = _tc_mlp([M2G_edge_structural], params["M2G_edge_embed"])

    g2 = _sc_gather_sum(P, Q, sidx2, ridx2, E2, K2, C)
    eG2M_1 = lax.slice(eG2M, (0, 0), (E1, H))
    eG2M_2 = lax.slice(eG2M, (E1, 0), (E, H))

    Nmp = ((Nm + 1280 - 1) // 1280) * 1280
    msg1 = _tc_mlp([eG2M_1], params["G2M_message"], residual=True,
                   block=4000, w1_offsets=[0], adds=[g1])
    part1 = _sc_scatter_add(msg1, ridx1, Nmp, K1, C)
    msg2 = _tc_mlp([eG2M_2], params["G2M_message"], residual=True,
                   block=4000, w1_offsets=[0], adds=[g2])
    part2 = _sc_scatter_add(msg2, ridx2, Nmp, K2, C, seed=part1)
    vG_new = _tc_mlp([vG], params["G_update"], residual=True)
    vM_new = _tc_mlp([vM, part2[0, :Nm], part2[1, :Nm]],
                     params["G2M_node_update"],
                     residual=True, w1_offsets=[0, H, H])

    return (vM_new[None], vG_new[None], eM2M, eM2G)


# speedup vs baseline: 1.0905x; 1.0363x over previous
import functools

import jax
import jax.numpy as jnp
from jax import lax
from jax.experimental import pallas as pl
from jax.experimental.pallas import tpu as pltpu
from jax.experimental.pallas import tpu_sc as plsc

H = 128
_NC = 2
_NS = 16
_NW = _NC * _NS


def _dot(a, b):
    return jnp.dot(a, b, preferred_element_type=jnp.float32,
                   precision=lax.Precision.DEFAULT)


def _make_mlp_body(n, m, residual, proj, block, nsplit):
    rows = block // nsplit

    def body(*refs):
        xs = refs[:n]
        adds = refs[n:n + m]
        w1s = refs[n + m:2 * n + m]
        rest = refs[2 * n + m:]
        if proj:
            b1, g, beta, w2, b2, wp, out, out2 = rest
        else:
            b1, g, beta, w2, b2, out = rest
        for s in range(nsplit):
            sl = pl.ds(s * rows, rows)
            acc = _dot(xs[0][sl, :], w1s[0][...])
            for i in range(1, n):
                acc = acc + _dot(xs[i][sl, :], w1s[i][...])
            for a in adds:
                acc = acc + a[sl, :]
            h = acc + b1[...][None]
            h = h * jax.nn.sigmoid(h)
            s1 = jnp.sum(h, axis=-1, keepdims=True)
            s2 = jnp.sum(h * h, axis=-1, keepdims=True)
            mu = s1 * (1.0 / H)
            var = s2 * (1.0 / H) - mu * mu
            t = lax.rsqrt(var + 1e-5)
            hn = (h - mu) * t * g[...][None] + beta[...][None]
            o = _dot(hn, w2[...]) + b2[...][None]
            if residual:
                o = xs[0][sl, :] + o
            out[sl, :] = o
            if proj:
                out2[sl, :] = _dot(o, wp[...])
    return body


def _tc_mlp(xs, p, residual=False, block=5000, w1_offsets=None,
            adds=(), extra_proj=None):
    n = len(xs)
    m = len(adds)
    N = xs[0].shape[0]
    assert N % block == 0, (N, block)
    if w1_offsets is None:
        w1_offsets = []
        off = 0
        for x in xs:
            w1_offsets.append(off)
            off += x.shape[1]
        assert off == p["W1"].shape[0]
    w1s = []
    for x, off in zip(xs, w1_offsets):
        d = x.shape[1]
        w1s.append(lax.slice(p["W1"], (off, 0), (off + d, H)))

    in_specs = []
    for x in xs:
        d = x.shape[1]
        in_specs.append(pl.BlockSpec((block, d), lambda i: (i, 0)))
    for _ in adds:
        in_specs.append(pl.BlockSpec((block, H), lambda i: (i, 0)))
    for x in xs:
        d = x.shape[1]
        in_specs.append(pl.BlockSpec((d, H), lambda i: (0, 0)))
    in_specs += [
        pl.BlockSpec((H,), lambda i: (0,)),
        pl.BlockSpec((H,), lambda i: (0,)),
        pl.BlockSpec((H,), lambda i: (0,)),
        pl.BlockSpec((H, H), lambda i: (0, 0)),
        pl.BlockSpec((H,), lambda i: (0,)),
    ]
    operands = [*xs, *adds, *w1s, p["b1"], p["g"], p["beta"], p["W2"],
                p["b2"]]
    if extra_proj is not None:
        in_specs.append(pl.BlockSpec((H, H), lambda i: (0, 0)))
        operands.append(extra_proj)
        out_specs = (pl.BlockSpec((block, H), lambda i: (i, 0)),
                     pl.BlockSpec((block, H), lambda i: (i, 0)))
        out_shape = (jax.ShapeDtypeStruct((N, H), jnp.float32),
                     jax.ShapeDtypeStruct((N, H), jnp.float32))
    else:
        out_specs = pl.BlockSpec((block, H), lambda i: (i, 0))
        out_shape = jax.ShapeDtypeStruct((N, H), jnp.float32)
    nsplit = 2 if block % 16 == 0 else 1
    return pl.pallas_call(
        _make_mlp_body(n, m, residual, extra_proj is not None, block, nsplit),
        grid=(N // block,),
        in_specs=in_specs,
        out_specs=out_specs,
        out_shape=out_shape,
        compiler_params=pltpu.CompilerParams(
            dimension_semantics=("arbitrary",)),
    )(*operands)


def _sc_gather_sum(p_tab, q_tab, sidx3, ridx3, E, K, C):
    mesh = plsc.VectorSubcoreMesh(core_axis_name="c", subcore_axis_name="s")

    def body(p_hbm, q_hbm, sidx_hbm, ridx_hbm, out_hbm,
             sidx_v, ridx_v, buf0, buf1, semp0, semp1, semq0, semq1):
        cid = lax.axis_index("c")
        sid = lax.axis_index("s")
        wid = sid * _NC + cid
        base = wid * (K * C)
        bufs = (buf0, buf1)
        semps = (semp0, semp1)
        semqs = (semq0, semq1)

        pltpu.sync_copy(sidx_hbm.at[wid], sidx_v)
        pltpu.sync_copy(ridx_hbm.at[wid], ridx_v)

        def start_p(j, slot):
            pltpu.async_copy(p_hbm.at[sidx_v.at[j]], bufs[slot], semps[slot])

        def wait_p(slot):
            pltpu.make_async_copy(p_hbm.at[sidx_v.at[0]], bufs[slot],
                                  semps[slot]).wait()

        def start_q(j, slot):
            pltpu.async_copy(q_hbm.at[ridx_v.at[j]], bufs[slot], semqs[slot],
                             add=True)

        def wait_q(slot):
            pltpu.make_async_copy(q_hbm.at[ridx_v.at[0]], bufs[slot],
                                  semqs[slot]).wait()

        def drain(j, slot):
            pltpu.sync_copy(bufs[slot], out_hbm.at[pl.ds(base + j * C, C)])

        start_p(0, 0)
        wait_p(0)
        start_q(0, 0)

        def pair(p, carry):
            a = 2 * p
            b = a + 1
            start_p(b, 1)
            wait_q(0)
            drain(a, 0)
            wait_p(1)
            start_q(b, 1)

            @pl.when(b + 1 < K)
            def _():
                start_p(b + 1, 0)

            wait_q(1)
            drain(b, 1)

            @pl.when(b + 1 < K)
            def _():
                wait_p(0)
                start_q(b + 1, 0)
            return carry
        lax.fori_loop(0, K // 2, pair, 0)
        if K % 2:
            wait_q(0)
            drain(K - 1, 0)

    k = pl.kernel(
        body,
        out_type=jax.ShapeDtypeStruct((E, H), jnp.float32),
        mesh=mesh,
        scratch_types=[
            pltpu.VMEM((K, C), jnp.int32),
            pltpu.VMEM((K, C), jnp.int32),
            pltpu.VMEM((C, H), jnp.float32),
            pltpu.VMEM((C, H), jnp.float32),
            pltpu.SemaphoreType.DMA,
            pltpu.SemaphoreType.DMA,
            pltpu.SemaphoreType.DMA,
            pltpu.SemaphoreType.DMA,
        ],
    )
    return k(p_tab, q_tab, sidx3, ridx3)


def _sc_scatter_add(msg, ridx3, Nmp, K, C, seed=None):
    mesh = plsc.VectorSubcoreMesh(core_axis_name="c", subcore_axis_name="s")
    zrows = Nmp // _NS
    ZB = 80
    assert zrows % ZB == 0 and zrows % 8 == 0

    def body(*refs):
        if seed is None:
            msg_hbm, ridx_hbm, out_hbm, idx_v, buf0, buf1, zbuf, acc, \
                sem0, sem1 = refs
        else:
            msg_hbm, ridx_hbm, seed_hbm, out_hbm, idx_v, buf0, buf1, acc, \
                sem0, sem1 = refs
        cid = lax.axis_index("c")
        sid = lax.axis_index("s")
        wid = sid * _NC + cid
        base = wid * (K * C)
        bufs = (buf0, buf1)
        sems = (sem0, sem1)

        pltpu.sync_copy(ridx_hbm.at[wid], idx_v)

        if seed is None:
            def zstep(r, carry):
                for c16 in range(H // 16):
                    zbuf[r, pl.ds(c16 * 16, 16)] = jnp.zeros((16,),
                                                             jnp.float32)
                return carry
            lax.fori_loop(0, ZB, zstep, 0)
            for t in range(zrows // ZB):
                pltpu.sync_copy(zbuf, acc.at[pl.ds(sid * zrows + t * ZB, ZB)])
        else:
            pltpu.sync_copy(seed_hbm.at[cid, pl.ds(sid * zrows, zrows)],
                            acc.at[pl.ds(sid * zrows, zrows)])
        plsc.subcore_barrier()

        def start(j, slot):
            pltpu.async_copy(msg_hbm.at[pl.ds(base + j * C, C)], bufs[slot],
                             sems[slot])

        def wait(slot):
            pltpu.make_async_copy(msg_hbm.at[pl.ds(base, C)], bufs[slot],
                                  sems[slot]).wait()

        def drain(j, slot):
            pltpu.sync_copy(bufs[slot], acc.at[idx_v.at[j]], add=True)

        start(0, 0)

        def pair(p, carry):
            a = 2 * p
            b = a + 1
            wait(0)
            start(b, 1)
            drain(a, 0)
            wait(1)

            @pl.when(b + 1 < K)
            def _():
                start(b + 1, 0)
            drain(b, 1)
            return carry
        lax.fori_loop(0, K // 2, pair, 0)
        if K % 2:
            wait(0)
            drain(K - 1, 0)
        plsc.subcore_barrier()

        pltpu.sync_copy(acc.at[pl.ds(sid * zrows, zrows)],
                        out_hbm.at[cid, pl.ds(sid * zrows, zrows)])

    scratch = [
        pltpu.VMEM((K, C), jnp.int32),
        pltpu.VMEM((C, H), jnp.float32),
        pltpu.VMEM((C, H), jnp.float32),
    ]
    if seed is None:
        scratch.append(pltpu.VMEM((ZB, H), jnp.float32))
    scratch += [
        pltpu.VMEM_SHARED((Nmp, H), jnp.float32),
        pltpu.SemaphoreType.DMA,
        pltpu.SemaphoreType.DMA,
    ]
    k = pl.kernel(
        body,
        out_type=jax.ShapeDtypeStruct((_NC, Nmp, H), jnp.float32),
        mesh=mesh,
        scratch_types=scratch,
    )
    if seed is None:
        return k(msg, ridx3)
    return k(msg, ridx3, seed)


def kernel(inputs, grid_structural, mesh_structural, M2M_edge_structural,
           G2M_edge_structural, M2G_edge_structural, senders, receivers,
           params):
    B_, Ng, _ = inputs.shape
    Nm = mesh_structural.shape[0]
    E = senders.shape[0]
    assert B_ == 1

    C = 40
    E1 = (E * 3 // 5) // (_NW * C) * (_NW * C)
    E2 = E - E1
    assert E2 % (_NW * C) == 0, (E, E1, E2)
    K1 = E1 // (_NW * C)
    K2 = E2 // (_NW * C)

    x_grid = inputs.reshape(Ng, inputs.shape[2])
    s32 = senders.astype(jnp.int32)
    r32 = receivers.astype(jnp.int32)
    sidx1 = lax.slice(s32, (0,), (E1,)).reshape(_NW, K1, C)
    ridx1 = lax.slice(r32, (0,), (E1,)).reshape(_NW, K1, C)
    sidx2 = lax.slice(s32, (E1,), (E,)).reshape(_NW, K2, C)
    ridx2 = lax.slice(r32, (E1,), (E,)).reshape(_NW, K2, C)

    w1_msg = params["G2M_message"]["W1"]
    w1_gs = lax.slice(w1_msg, (H, 0), (2 * H, H))
    w1_ms = lax.slice(w1_msg, (2 * H, 0), (3 * H, H))
    vG, P = _tc_mlp([x_grid, grid_structural], params["grid_node_embed"],
                    extra_proj=w1_gs)
    vM, Q = _tc_mlp([mesh_structural], params["mesh_node_embed"],
                    extra_proj=w1_ms)

    g1 = _sc_gather_sum(P, Q, sidx1, ridx1, E1, K1, C)
    eG2M = _tc_mlp([G2M_edge_structural], params["G2M_edge_embed"])
    eM2M = _tc_mlp([M2M_edge_structural], params["M2M_edge_embed"])
    eM2G = _tc_mlp([M2G_edge_structural], params["M2G_edge_embed"])

    g2 = _sc_gather_sum(P, Q, sidx2, ridx2, E2, K2, C)
    eG2M_1 = lax.slice(eG2M, (0, 0), (E1, H))
    eG2M_2 = lax.slice(eG2M, (E1, 0), (E, H))

    Nmp = ((Nm + 1280 - 1) // 1280) * 1280
    msg1 = _tc_mlp([eG2M_1], params["G2M_message"], residual=True,
                   block=4000, w1_offsets=[0], adds=[g1])
    part1 = _sc_scatter_add(msg1, ridx1, Nmp, K1, C)
    msg2 = _tc_mlp([eG2M_2], params["G2M_message"], residual=True,
                   block=4000, w1_offsets=[0], adds=[g2])
    part2 = _sc_scatter_add(msg2, ridx2, Nmp, K2, C)
    vG_new = _tc_mlp([vG], params["G_update"], residual=True)
    vM_new = _tc_mlp([vM, part1[0, :Nm], part1[1, :Nm],
                      part2[0, :Nm], part2[1, :Nm]],
                     params["G2M_node_update"],
                     residual=True, w1_offsets=[0, H, H, H, H])

    return (vM_new[None], vG_new[None], eM2M, eM2G)


# speedup vs baseline: 1.1057x; 1.0140x over previous
import functools

import jax
import jax.numpy as jnp
from jax import lax
from jax.experimental import pallas as pl
from jax.experimental.pallas import tpu as pltpu
from jax.experimental.pallas import tpu_sc as plsc

H = 128
_NC = 2
_NS = 16
_NW = _NC * _NS


def _dot(a, b):
    return jnp.dot(a, b, preferred_element_type=jnp.float32,
                   precision=lax.Precision.DEFAULT)


def _make_mlp_body(n, m, residual, proj, block, nsplit):
    rows = block // nsplit

    def body(*refs):
        xs = refs[:n]
        adds = refs[n:n + m]
        w1s = refs[n + m:2 * n + m]
        rest = refs[2 * n + m:]
        if proj:
            b1, g, beta, w2, b2, wp, out, out2 = rest
        else:
            b1, g, beta, w2, b2, out = rest
        for s in range(nsplit):
            sl = pl.ds(s * rows, rows)
            acc = _dot(xs[0][sl, :], w1s[0][...])
            for i in range(1, n):
                acc = acc + _dot(xs[i][sl, :], w1s[i][...])
            for a in adds:
                acc = acc + a[sl, :]
            h = acc + b1[...][None]
            h = h * jax.nn.sigmoid(h)
            s1 = jnp.sum(h, axis=-1, keepdims=True)
            s2 = jnp.sum(h * h, axis=-1, keepdims=True)
            mu = s1 * (1.0 / H)
            var = s2 * (1.0 / H) - mu * mu
            t = lax.rsqrt(var + 1e-5)
            hn = (h - mu) * t * g[...][None] + beta[...][None]
            o = _dot(hn, w2[...]) + b2[...][None]
            if residual:
                o = xs[0][sl, :] + o
            out[sl, :] = o
            if proj:
                out2[sl, :] = _dot(o, wp[...])
    return body


def _tc_mlp(xs, p, residual=False, block=5000, w1_offsets=None,
            adds=(), extra_proj=None):
    n = len(xs)
    m = len(adds)
    N = xs[0].shape[0]
    assert N % block == 0, (N, block)
    if w1_offsets is None:
        w1_offsets = []
        off = 0
        for x in xs:
            w1_offsets.append(off)
            off += x.shape[1]
        assert off == p["W1"].shape[0]
    w1s = []
    for x, off in zip(xs, w1_offsets):
        d = x.shape[1]
        w1s.append(lax.slice(p["W1"], (off, 0), (off + d, H)))

    in_specs = []
    for x in xs:
        d = x.shape[1]
        in_specs.append(pl.BlockSpec((block, d), lambda i: (i, 0)))
    for _ in adds:
        in_specs.append(pl.BlockSpec((block, H), lambda i: (i, 0)))
    for x in xs:
        d = x.shape[1]
        in_specs.append(pl.BlockSpec((d, H), lambda i: (0, 0)))
    in_specs += [
        pl.BlockSpec((H,), lambda i: (0,)),
        pl.BlockSpec((H,), lambda i: (0,)),
        pl.BlockSpec((H,), lambda i: (0,)),
        pl.BlockSpec((H, H), lambda i: (0, 0)),
        pl.BlockSpec((H,), lambda i: (0,)),
    ]
    operands = [*xs, *adds, *w1s, p["b1"], p["g"], p["beta"], p["W2"],
                p["b2"]]
    if extra_proj is not None:
        in_specs.append(pl.BlockSpec((H, H), lambda i: (0, 0)))
        operands.append(extra_proj)
        out_specs = (pl.BlockSpec((block, H), lambda i: (i, 0)),
                     pl.BlockSpec((block, H), lambda i: (i, 0)))
        out_shape = (jax.ShapeDtypeStruct((N, H), jnp.float32),
                     jax.ShapeDtypeStruct((N, H), jnp.float32))
    else:
        out_specs = pl.BlockSpec((block, H), lambda i: (i, 0))
        out_shape = jax.ShapeDtypeStruct((N, H), jnp.float32)
    nsplit = 2 if block % 16 == 0 else 1
    return pl.pallas_call(
        _make_mlp_body(n, m, residual, extra_proj is not None, block, nsplit),
        grid=(N // block,),
        in_specs=in_specs,
        out_specs=out_specs,
        out_shape=out_shape,
        compiler_params=pltpu.CompilerParams(
            dimension_semantics=("arbitrary",)),
    )(*operands)


def _sc_gather_sum(p_tab, q_tab, sidx3, ridx3, E, K, C):
    mesh = plsc.VectorSubcoreMesh(core_axis_name="c", subcore_axis_name="s")
    Nqp = q_tab.shape[0]
    qrows = Nqp // _NS
    assert qrows * _NS == Nqp and qrows % 8 == 0

    def body(p_hbm, q_hbm, sidx_hbm, ridx_hbm, out_hbm,
             sidx_v, ridx_v, buf0, buf1, qsp, semp0, semp1, semq0, semq1):
        cid = lax.axis_index("c")
        sid = lax.axis_index("s")
        wid = sid * _NC + cid
        base = wid * (K * C)
        bufs = (buf0, buf1)
        semps = (semp0, semp1)
        semqs = (semq0, semq1)

        pltpu.sync_copy(q_hbm.at[pl.ds(sid * qrows, qrows)],
                        qsp.at[pl.ds(sid * qrows, qrows)])
        pltpu.sync_copy(sidx_hbm.at[wid], sidx_v)
        pltpu.sync_copy(ridx_hbm.at[wid], ridx_v)
        plsc.subcore_barrier()

        def start_p(j, slot):
            pltpu.async_copy(p_hbm.at[sidx_v.at[j]], bufs[slot], semps[slot])

        def wait_p(slot):
            pltpu.make_async_copy(p_hbm.at[sidx_v.at[0]], bufs[slot],
                                  semps[slot]).wait()

        def start_q(j, slot):
            pltpu.async_copy(qsp.at[ridx_v.at[j]], bufs[slot], semqs[slot],
                             add=True)

        def wait_q(slot):
            pltpu.make_async_copy(qsp.at[ridx_v.at[0]], bufs[slot],
                                  semqs[slot]).wait()

        def drain(j, slot):
            pltpu.sync_copy(bufs[slot], out_hbm.at[pl.ds(base + j * C, C)])

        start_p(0, 0)
        wait_p(0)
        start_q(0, 0)

        def pair(p, carry):
            a = 2 * p
            b = a + 1
            start_p(b, 1)
            wait_q(0)
            drain(a, 0)
            wait_p(1)
            start_q(b, 1)

            @pl.when(b + 1 < K)
            def _():
                start_p(b + 1, 0)

            wait_q(1)
            drain(b, 1)

            @pl.when(b + 1 < K)
            def _():
                wait_p(0)
                start_q(b + 1, 0)
            return carry
        lax.fori_loop(0, K // 2, pair, 0)
        if K % 2:
            wait_q(0)
            drain(K - 1, 0)

    k = pl.kernel(
        body,
        out_type=jax.ShapeDtypeStruct((E, H), jnp.float32),
        mesh=mesh,
        scratch_types=[
            pltpu.VMEM((K, C), jnp.int32),
            pltpu.VMEM((K, C), jnp.int32),
            pltpu.VMEM((C, H), jnp.float32),
            pltpu.VMEM((C, H), jnp.float32),
            pltpu.VMEM_SHARED((Nqp, H), jnp.float32),
            pltpu.SemaphoreType.DMA,
            pltpu.SemaphoreType.DMA,
            pltpu.SemaphoreType.DMA,
            pltpu.SemaphoreType.DMA,
        ],
    )
    return k(p_tab, q_tab, sidx3, ridx3)


def _sc_scatter_add(msg, ridx3, Nmp, K, C, seed=None):
    mesh = plsc.VectorSubcoreMesh(core_axis_name="c", subcore_axis_name="s")
    zrows = Nmp // _NS
    ZB = 80
    assert zrows % ZB == 0 and zrows % 8 == 0

    def body(*refs):
        if seed is None:
            msg_hbm, ridx_hbm, out_hbm, idx_v, buf0, buf1, zbuf, acc, \
                sem0, sem1 = refs
        else:
            msg_hbm, ridx_hbm, seed_hbm, out_hbm, idx_v, buf0, buf1, acc, \
                sem0, sem1 = refs
        cid = lax.axis_index("c")
        sid = lax.axis_index("s")
        wid = sid * _NC + cid
        base = wid * (K * C)
        bufs = (buf0, buf1)
        sems = (sem0, sem1)

        pltpu.sync_copy(ridx_hbm.at[wid], idx_v)

        if seed is None:
            def zstep(r, carry):
                for c16 in range(H // 16):
                    zbuf[r, pl.ds(c16 * 16, 16)] = jnp.zeros((16,),
                                                             jnp.float32)
                return carry
            lax.fori_loop(0, ZB, zstep, 0)
            for t in range(zrows // ZB):
                pltpu.sync_copy(zbuf, acc.at[pl.ds(sid * zrows + t * ZB, ZB)])
        else:
            pltpu.sync_copy(seed_hbm.at[cid, pl.ds(sid * zrows, zrows)],
                            acc.at[pl.ds(sid * zrows, zrows)])
        plsc.subcore_barrier()

        def start(j, slot):
            pltpu.async_copy(msg_hbm.at[pl.ds(base + j * C, C)], bufs[slot],
                             sems[slot])

        def wait(slot):
            pltpu.make_async_copy(msg_hbm.at[pl.ds(base, C)], bufs[slot],
                                  sems[slot]).wait()

        def drain(j, slot):
            pltpu.sync_copy(bufs[slot], acc.at[idx_v.at[j]], add=True)

        start(0, 0)

        def pair(p, carry):
            a = 2 * p
            b = a + 1
            wait(0)
            start(b, 1)
            drain(a, 0)
            wait(1)

            @pl.when(b + 1 < K)
            def _():
                start(b + 1, 0)
            drain(b, 1)
            return carry
        lax.fori_loop(0, K // 2, pair, 0)
        if K % 2:
            wait(0)
            drain(K - 1, 0)
        plsc.subcore_barrier()

        pltpu.sync_copy(acc.at[pl.ds(sid * zrows, zrows)],
                        out_hbm.at[cid, pl.ds(sid * zrows, zrows)])

    scratch = [
        pltpu.VMEM((K, C), jnp.int32),
        pltpu.VMEM((C, H), jnp.float32),
        pltpu.VMEM((C, H), jnp.float32),
    ]
    if seed is None:
        scratch.append(pltpu.VMEM((ZB, H), jnp.float32))
    scratch += [
        pltpu.VMEM_SHARED((Nmp, H), jnp.float32),
        pltpu.SemaphoreType.DMA,
        pltpu.SemaphoreType.DMA,
    ]
    k = pl.kernel(
        body,
        out_type=jax.ShapeDtypeStruct((_NC, Nmp, H), jnp.float32),
        mesh=mesh,
        scratch_types=scratch,
    )
    if seed is None:
        return k(msg, ridx3)
    return k(msg, ridx3, seed)


def kernel(inputs, grid_structural, mesh_structural, M2M_edge_structural,
           G2M_edge_structural, M2G_edge_structural, senders, receivers,
           params):
    B_, Ng, _ = inputs.shape
    Nm = mesh_structural.shape[0]
    E = senders.shape[0]
    assert B_ == 1

    C = 40
    E1 = (E * 3 // 5) // (_NW * C) * (_NW * C)
    E2 = E - E1
    assert E2 % (_NW * C) == 0, (E, E1, E2)
    K1 = E1 // (_NW * C)
    K2 = E2 // (_NW * C)

    x_grid = inputs.reshape(Ng, inputs.shape[2])
    s32 = senders.astype(jnp.int32)
    r32 = receivers.astype(jnp.int32)
    sidx1 = lax.slice(s32, (0,), (E1,)).reshape(_NW, K1, C)
    ridx1 = lax.slice(r32, (0,), (E1,)).reshape(_NW, K1, C)
    sidx2 = lax.slice(s32, (E1,), (E,)).reshape(_NW, K2, C)
    ridx2 = lax.slice(r32, (E1,), (E,)).reshape(_NW, K2, C)

    w1_msg = params["G2M_message"]["W1"]
    w1_gs = lax.slice(w1_msg, (H, 0), (2 * H, H))
    w1_ms = lax.slice(w1_msg, (2 * H, 0), (3 * H, H))
    Nmp = ((Nm + 1280 - 1) // 1280) * 1280
    vG, P = _tc_mlp([x_grid, grid_structural], params["grid_node_embed"],
                    extra_proj=w1_gs)
    mesh_pad = jnp.pad(mesh_structural, ((0, Nmp - Nm), (0, 0)))
    vMp, Q = _tc_mlp([mesh_pad], params["mesh_node_embed"],
                     extra_proj=w1_ms, block=5120)
    vM = lax.slice(vMp, (0, 0), (Nm, H))

    g1 = _sc_gather_sum(P, Q, sidx1, ridx1, E1, K1, C)
    eG2M = _tc_mlp([G2M_edge_structural], params["G2M_edge_embed"])
    eM2M = _tc_mlp([M2M_edge_structural], params["M2M_edge_embed"])
    eM2G = _tc_mlp([M2G_edge_structural], params["M2G_edge_embed"])

    g2 = _sc_gather_sum(P, Q, sidx2, ridx2, E2, K2, C)
    eG2M_1 = lax.slice(eG2M, (0, 0), (E1, H))
    eG2M_2 = lax.slice(eG2M, (E1, 0), (E, H))

    msg1 = _tc_mlp([eG2M_1], params["G2M_message"], residual=True,
                   block=4000, w1_offsets=[0], adds=[g1])
    part1 = _sc_scatter_add(msg1, ridx1, Nmp, K1, C)
    msg2 = _tc_mlp([eG2M_2], params["G2M_message"], residual=True,
                   block=4000, w1_offsets=[0], adds=[g2])
    part2 = _sc_scatter_add(msg2, ridx2, Nmp, K2, C)
    vG_new = _tc_mlp([vG], params["G_update"], residual=True)
    vM_new = _tc_mlp([vM, part1[0, :Nm], part1[1, :Nm],
                      part2[0, :Nm], part2[1, :Nm]],
                     params["G2M_node_update"],
                     residual=True, w1_offsets=[0, H, H, H, H])

    return (vM_new[None], vG_new[None], eM2M, eM2G)
